# baseline (speedup 1.0000x reference)
import jax
import jax.numpy as jnp
from jax import lax
from jax.experimental import pallas as pl
from jax.experimental.pallas import tpu as pltpu
from jax.experimental.pallas import tpu_sc as plsc

EMBED = 8
LANES = 16
PACK = 16
ROW = PACK * EMBED
NUM_CORES = 2
NUM_SUBCORES = 16
NUM_WORKERS = NUM_CORES * NUM_SUBCORES


def _dot_body(chunk, half, users_hbm, items_hbm, ut_hbm, it_hbm, out_hbm,
              idx_u, idx_i, rid_u, rid_i, gu, gi, out_v, sem_u, sem_i):
    wid = lax.axis_index("s") * NUM_CORES + lax.axis_index("c")
    base = wid * chunk
    pltpu.sync_copy(users_hbm.at[pl.ds(base, chunk)], idx_u)
    pltpu.sync_copy(items_hbm.at[pl.ds(base, chunk)], idx_i)

    for g in range(chunk // LANES):
        s = pl.ds(g * LANES, LANES)
        rid_u[s] = idx_u[s] >> 4
        rid_i[s] = idx_i[s] >> 4

    lane = lax.iota(jnp.int32, LANES)
    for h in range(chunk // half):
        hbase = h * half
        cp_u = pltpu.async_copy(
            ut_hbm.at[rid_u.at[pl.ds(hbase, half)]], gu, sem_u)
        cp_i = pltpu.async_copy(
            it_hbm.at[rid_i.at[pl.ds(hbase, half)]], gi, sem_i)
        cp_u.wait()
        cp_i.wait()
        for g in range(half // LANES):
            s = pl.ds(hbase + g * LANES, LANES)
            cu = (idx_u[s] & 15) * EMBED
            ci = (idx_i[s] & 15) * EMBED
            row = lane + g * LANES
            acc = None
            for d in range(EMBED):
                u = plsc.load_gather(gu, [row, cu + d])
                v = plsc.load_gather(gi, [row, ci + d])
                acc = u * v if acc is None else acc + u * v
            out_v[s] = acc

    pltpu.sync_copy(out_v, out_hbm.at[pl.ds(base, chunk)])


def kernel(users, items, user_table, item_table):
    batch = users.shape[0]
    chunk = batch // NUM_WORKERS
    half = chunk // 2
    ut = user_table.reshape(-1, ROW)
    it = item_table.reshape(-1, ROW)
    mesh = plsc.VectorSubcoreMesh(core_axis_name="c", subcore_axis_name="s")

    def body(*refs):
        _dot_body(chunk, half, *refs)

    k = pl.kernel(
        body,
        mesh=mesh,
        compiler_params=pltpu.CompilerParams(needs_layout_passes=False),
        out_type=jax.ShapeDtypeStruct((batch,), jnp.float32),
        scratch_types=[
            pltpu.VMEM((chunk,), jnp.int32),
            pltpu.VMEM((chunk,), jnp.int32),
            pltpu.VMEM((chunk,), jnp.int32),
            pltpu.VMEM((chunk,), jnp.int32),
            pltpu.VMEM((half, ROW), jnp.float32),
            pltpu.VMEM((half, ROW), jnp.float32),
            pltpu.VMEM((chunk,), jnp.float32),
            pltpu.SemaphoreType.DMA,
            pltpu.SemaphoreType.DMA,
        ],
    )
    return k(users.astype(jnp.int32), items.astype(jnp.int32), ut, it)
